# Initial kernel scaffold; baseline (speedup 1.0000x reference)
import jax
import jax.numpy as jnp
from jax.experimental import pallas as pl


def kernel(dense_t1, dense_t2, t1_mask, t2_mask, W_gat, att_src, att_dst, b_gat, Wd1, bd1, Wd2, bd2, Wi1, bi1, Wi2, bi2, emb_del, emb_ins):
    raise NotImplementedError("write your pallas kernel here")


# speedup vs baseline: 22.3113x; 22.3113x over previous
import jax
import jax.numpy as jnp
from jax.experimental import pallas as pl
from jax.experimental.pallas import tpu as pltpu

_B = 2
_N = 512
_D = 128


def _soft_hd_kernel(p1_ref, p2_ref, W_ref, asrc_ref, adst_ref, bgat_ref,
                    Wd1_ref, bd1_ref, Wd2_ref, bd2_ref,
                    Wi1_ref, bi1_ref, Wi2_ref, bi2_ref,
                    edel_ref, eins_ref, out_ref):
    p1 = p1_ref[0]
    p2 = p2_ref[0]
    W = W_ref[...]
    asrc = asrc_ref[...]
    adst = adst_ref[...]
    bgat = bgat_ref[...]

    idx = jax.lax.broadcasted_iota(jnp.int32, (_N, 1), 0)
    has_m = (idx >= 1).astype(jnp.float32)
    has_p = (idx <= _N - 2).astype(jnp.float32)
    neg_big = jnp.float32(-1e30)

    def lrelu(x):
        return jnp.where(x >= 0, x, 0.2 * x)

    def gat(p):
        h = jnp.dot(p, W.T, preferred_element_type=jnp.float32)
        es = jnp.dot(h, asrc, preferred_element_type=jnp.float32)
        ed = jnp.dot(h, adst, preferred_element_type=jnp.float32)
        es_m = jnp.roll(es, 1, axis=0)
        es_p = jnp.roll(es, -1, axis=0)
        e_m = jnp.where(has_m > 0, lrelu(es_m + ed), neg_big)
        e_s = lrelu(es + ed)
        e_p = jnp.where(has_p > 0, lrelu(es_p + ed), neg_big)
        m = jnp.maximum(jnp.maximum(e_m, e_p), e_s)
        w_m = jnp.exp(e_m - m) * has_m
        w_s = jnp.exp(e_s - m)
        w_p = jnp.exp(e_p - m) * has_p
        inv_s = 1.0 / (w_m + w_s + w_p)
        h_m = jnp.roll(h, 1, axis=0)
        h_p = jnp.roll(h, -1, axis=0)
        agg = w_m * h_m + w_s * h + w_p * h_p
        return agg * inv_s + bgat

    h1 = gat(p1)
    h2 = gat(p2)

    def sqd(a, b):
        aa = jnp.sum(a * a, axis=1, keepdims=True)
        bb = jnp.sum(b * b, axis=1, keepdims=True)
        bb_t = jax.lax.transpose(bb, (1, 0))
        g = jnp.dot(a, b.T, preferred_element_type=jnp.float32)
        return jnp.maximum(aa + bb_t - 2.0 * g, 0.0)

    dm = 0.5 * (sqd(p1, p2) + sqd(h1, h2))

    def mlp_abs(p, W1, b1, W2, b2):
        q = jnp.maximum(
            jnp.dot(p, W1.T, preferred_element_type=jnp.float32) + b1, 0.0)
        r = jnp.sum(q * W2, axis=1, keepdims=True)
        return jnp.abs(r + b2)

    mean_del = jnp.sum(edel_ref[...]) / 5.0
    mean_ins = jnp.sum(eins_ref[...]) / 5.0
    d1 = mean_del + mlp_abs(p1, Wd1_ref[...], bd1_ref[...],
                            Wd2_ref[...], bd2_ref[0, 0])
    d2 = mean_ins + mlp_abs(p2, Wi1_ref[...], bi1_ref[...],
                            Wi2_ref[...], bi2_ref[0, 0])

    colmin = jnp.min(dm, axis=0, keepdims=True)
    rowmin = jnp.min(dm, axis=1, keepdims=True)
    a_v = jnp.minimum(colmin, jax.lax.transpose(d2, (1, 0)))
    b_v = jnp.minimum(rowmin, d1)
    total = (jnp.sum(a_v) + jnp.sum(b_v)) / jnp.float32(2 * _N)
    out_ref[...] = jnp.full((1, 1, 128), total, dtype=jnp.float32)


def kernel(dense_t1, dense_t2, t1_mask, t2_mask, W_gat, att_src, att_dst,
           b_gat, Wd1, bd1, Wd2, bd2, Wi1, bi1, Wi2, bi2, emb_del, emb_ins):
    del t1_mask, t2_mask
    asrc = att_src.reshape(_D, 1)
    adst = att_dst.reshape(_D, 1)
    bgat = b_gat.reshape(1, _D)
    bd1r = bd1.reshape(1, _D // 2)
    bd2r = bd2.reshape(1, 1)
    bi1r = bi1.reshape(1, _D // 2)
    bi2r = bi2.reshape(1, 1)
    edel = jnp.zeros((1, 128), jnp.float32).at[0, :5].set(emb_del[_N - 5:_N, 0])
    eins = jnp.zeros((1, 128), jnp.float32).at[0, :5].set(emb_ins[_N - 5:_N, 0])

    def fixed(shape):
        nd = len(shape)
        return pl.BlockSpec(shape, lambda b: (0,) * nd)

    out = pl.pallas_call(
        _soft_hd_kernel,
        grid=(_B,),
        in_specs=[
            pl.BlockSpec((1, _N, _D), lambda b: (b, 0, 0)),
            pl.BlockSpec((1, _N, _D), lambda b: (b, 0, 0)),
            fixed((_D, _D)),
            fixed((_D, 1)),
            fixed((_D, 1)),
            fixed((1, _D)),
            fixed((_D // 2, _D)),
            fixed((1, _D // 2)),
            fixed((1, _D // 2)),
            pl.BlockSpec(memory_space=pltpu.SMEM),
            fixed((_D // 2, _D)),
            fixed((1, _D // 2)),
            fixed((1, _D // 2)),
            pl.BlockSpec(memory_space=pltpu.SMEM),
            fixed((1, 128)),
            fixed((1, 128)),
        ],
        out_specs=pl.BlockSpec((1, 1, 128), lambda b: (b, 0, 0)),
        out_shape=jax.ShapeDtypeStruct((_B, 1, 128), jnp.float32),
        compiler_params=pltpu.CompilerParams(
            dimension_semantics=("parallel",)),
    )(dense_t1, dense_t2, W_gat, asrc, adst, bgat,
      Wd1, bd1r, Wd2, bd2r, Wi1, bi1r, Wi2, bi2r, edel, eins)
    return out[:, 0, 0]


# speedup vs baseline: 24.0888x; 1.0797x over previous
import jax
import jax.numpy as jnp
from jax.experimental import pallas as pl
from jax.experimental.pallas import tpu as pltpu

_B = 2
_N = 512
_D = 128


def _soft_hd_kernel(p1_ref, p2_ref, W_ref, asrc_ref, adst_ref, bgat_ref,
                    Wd1_ref, bd1_ref, Wd2_ref, bd2_ref,
                    Wi1_ref, bi1_ref, Wi2_ref, bi2_ref,
                    edel_ref, eins_ref, out_ref):
    P1 = p1_ref[...]
    P2 = p2_ref[...]
    W = W_ref[...]
    asrc = asrc_ref[...]
    adst = adst_ref[...]
    bgat = bgat_ref[...]

    M = _B * _N
    idx = jax.lax.broadcasted_iota(jnp.int32, (M, 1), 0)
    pos = jax.lax.bitwise_and(idx, _N - 1)
    has_m = (pos >= 1).astype(jnp.float32)
    has_p = (pos <= _N - 2).astype(jnp.float32)
    neg_big = jnp.float32(-1e30)

    def lrelu(x):
        return jnp.where(x >= 0, x, 0.2 * x)

    def gat(p):
        h = jnp.dot(p, W.T, preferred_element_type=jnp.float32)
        es = jnp.sum(h * asrc, axis=1, keepdims=True)
        ed = jnp.sum(h * adst, axis=1, keepdims=True)
        es_m = jnp.roll(es, 1, axis=0)
        es_p = jnp.roll(es, -1, axis=0)
        e_m = jnp.where(has_m > 0, lrelu(es_m + ed), neg_big)
        e_s = lrelu(es + ed)
        e_p = jnp.where(has_p > 0, lrelu(es_p + ed), neg_big)
        m = jnp.maximum(jnp.maximum(e_m, e_p), e_s)
        w_m = jnp.exp(e_m - m) * has_m
        w_s = jnp.exp(e_s - m)
        w_p = jnp.exp(e_p - m) * has_p
        inv_s = 1.0 / (w_m + w_s + w_p)
        h_m = jnp.roll(h, 1, axis=0)
        h_p = jnp.roll(h, -1, axis=0)
        agg = w_m * h_m + w_s * h + w_p * h_p
        return agg * inv_s + bgat

    H1 = gat(P1)
    H2 = gat(P2)

    aa_p = jnp.sum(P1 * P1, axis=1, keepdims=True)
    bb_p = jnp.sum(P2 * P2, axis=1, keepdims=True)
    aa_h = jnp.sum(H1 * H1, axis=1, keepdims=True)
    bb_h = jnp.sum(H2 * H2, axis=1, keepdims=True)

    def mlp_scores(p, W1_ref, b1_ref, W2_ref, b2_ref, mean_ref):
        q = jnp.maximum(
            jnp.dot(p, W1_ref[...].T, preferred_element_type=jnp.float32)
            + b1_ref[...], 0.0)
        r = jnp.sum(q * W2_ref[...], axis=1, keepdims=True)
        mean = jnp.sum(mean_ref[...] * _emb_mask()) / 5.0
        return mean + jnp.abs(r + b2_ref[0])

    d1 = mlp_scores(P1, Wd1_ref, bd1_ref, Wd2_ref, bd2_ref, edel_ref)
    d2 = mlp_scores(P2, Wi1_ref, bi1_ref, Wi2_ref, bi2_ref, eins_ref)

    for b in range(_B):
        lo, hi = b * _N, (b + 1) * _N
        a1 = P1[lo:hi, :]
        b1 = P2[lo:hi, :]
        h1 = H1[lo:hi, :]
        h2 = H2[lo:hi, :]
        g_w = jnp.dot(a1, b1.T, preferred_element_type=jnp.float32)
        g_c = jnp.dot(h1, h2.T, preferred_element_type=jnp.float32)
        aw = aa_p[lo:hi, :]
        bw = jax.lax.transpose(bb_p[lo:hi, :], (1, 0))
        ac = aa_h[lo:hi, :]
        bc = jax.lax.transpose(bb_h[lo:hi, :], (1, 0))
        dm2 = (jnp.maximum(aw + bw - 2.0 * g_w, 0.0)
               + jnp.maximum(ac + bc - 2.0 * g_c, 0.0))
        colmin = jnp.min(dm2, axis=0, keepdims=True)
        rowmin = jnp.min(dm2, axis=1, keepdims=True)
        a_v = jnp.minimum(colmin, jax.lax.transpose(2.0 * d2[lo:hi, :], (1, 0)))
        b_v = jnp.minimum(rowmin, 2.0 * d1[lo:hi, :])
        out_ref[b] = (jnp.sum(a_v) + jnp.sum(b_v)) / jnp.float32(4 * _N)


def _emb_mask():
    i = jax.lax.broadcasted_iota(jnp.int32, (8, 1), 0)
    return (i >= 3).astype(jnp.float32)


def kernel(dense_t1, dense_t2, t1_mask, t2_mask, W_gat, att_src, att_dst,
           b_gat, Wd1, bd1, Wd2, bd2, Wi1, bi1, Wi2, bi2, emb_del, emb_ins):
    del t1_mask, t2_mask
    P1 = dense_t1.reshape(_B * _N, _D)
    P2 = dense_t2.reshape(_B * _N, _D)
    asrc = att_src.reshape(1, _D)
    adst = att_dst.reshape(1, _D)
    bgat = b_gat.reshape(1, _D)
    bd1r = bd1.reshape(1, _D // 2)
    bi1r = bi1.reshape(1, _D // 2)

    def fixed(shape):
        return pl.BlockSpec(shape, lambda i: (0,) * len(shape))

    out = pl.pallas_call(
        _soft_hd_kernel,
        grid=(1,),
        in_specs=[
            fixed((_B * _N, _D)),
            fixed((_B * _N, _D)),
            fixed((_D, _D)),
            fixed((1, _D)),
            fixed((1, _D)),
            fixed((1, _D)),
            fixed((_D // 2, _D)),
            fixed((1, _D // 2)),
            fixed((1, _D // 2)),
            pl.BlockSpec(memory_space=pltpu.SMEM),
            fixed((_D // 2, _D)),
            fixed((1, _D // 2)),
            fixed((1, _D // 2)),
            pl.BlockSpec(memory_space=pltpu.SMEM),
            pl.BlockSpec((8, 1), lambda i: (_N // 8 - 1, 0)),
            pl.BlockSpec((8, 1), lambda i: (_N // 8 - 1, 0)),
        ],
        out_specs=pl.BlockSpec(memory_space=pltpu.SMEM),
        out_shape=jax.ShapeDtypeStruct((_B,), jnp.float32),
    )(P1, P2, W_gat, asrc, adst, bgat,
      Wd1, bd1r, Wd2, bd2, Wi1, bi1r, Wi2, bi2, emb_del, emb_ins)
    return out


# speedup vs baseline: 25.1686x; 1.0448x over previous
import jax
import jax.numpy as jnp
from jax.experimental import pallas as pl
from jax.experimental.pallas import tpu as pltpu

_B = 2
_N = 512
_D = 128


def _soft_hd_kernel(p1_ref, p2_ref, W_ref, asrc_ref, adst_ref, bgat_ref,
                    Wd1_ref, bd1_ref, Wd2_ref, bd2_ref,
                    Wi1_ref, bi1_ref, Wi2_ref, bi2_ref,
                    edel_ref, eins_ref, out_ref):
    P1 = p1_ref[...]
    P2 = p2_ref[...]
    W = W_ref[...]
    asrc = asrc_ref[...]
    adst = adst_ref[...]
    bgat = bgat_ref[...]

    M = _B * _N
    idx = jax.lax.broadcasted_iota(jnp.int32, (M, 1), 0)
    pos = jax.lax.bitwise_and(idx, _N - 1)
    has_m = (pos >= 1).astype(jnp.float32)
    has_p = (pos <= _N - 2).astype(jnp.float32)
    neg_big = jnp.float32(-1e30)

    def lrelu(x):
        return jnp.where(x >= 0, x, 0.2 * x)

    def gat(p):
        h = jnp.dot(p, W.T, preferred_element_type=jnp.float32)
        es = jnp.sum(h * asrc, axis=1, keepdims=True)
        ed = jnp.sum(h * adst, axis=1, keepdims=True)
        es_m = jnp.roll(es, 1, axis=0)
        es_p = jnp.roll(es, -1, axis=0)
        e_m = jnp.where(has_m > 0, lrelu(es_m + ed), neg_big)
        e_s = lrelu(es + ed)
        e_p = jnp.where(has_p > 0, lrelu(es_p + ed), neg_big)
        m = jnp.maximum(jnp.maximum(e_m, e_p), e_s)
        w_m = jnp.exp(e_m - m) * has_m
        w_s = jnp.exp(e_s - m)
        w_p = jnp.exp(e_p - m) * has_p
        inv_s = 1.0 / (w_m + w_s + w_p)
        h_m = jnp.roll(h, 1, axis=0)
        h_p = jnp.roll(h, -1, axis=0)
        agg = w_m * h_m + w_s * h + w_p * h_p
        return agg * inv_s + bgat

    H1 = gat(P1)
    H2 = gat(P2)

    U = jnp.concatenate([-2.0 * P1, -2.0 * H1], axis=1)
    V = jnp.concatenate([P2, H2], axis=1)
    aa_t = (jnp.sum(P1 * P1, axis=1, keepdims=True)
            + jnp.sum(H1 * H1, axis=1, keepdims=True))
    bb_t = (jnp.sum(P2 * P2, axis=1, keepdims=True)
            + jnp.sum(H2 * H2, axis=1, keepdims=True))

    def mlp_scores(p, W1_ref, b1_ref, W2_ref, b2_ref, mean_ref):
        q = jnp.maximum(
            jnp.dot(p, W1_ref[...].T, preferred_element_type=jnp.float32)
            + b1_ref[...], 0.0)
        r = jnp.sum(q * W2_ref[...], axis=1, keepdims=True)
        mean = jnp.sum(mean_ref[...] * _emb_mask()) / 5.0
        return mean + jnp.abs(r + b2_ref[0])

    d1 = mlp_scores(P1, Wd1_ref, bd1_ref, Wd2_ref, bd2_ref, edel_ref)
    d2 = mlp_scores(P2, Wi1_ref, bi1_ref, Wi2_ref, bi2_ref, eins_ref)

    for b in range(_B):
        lo, hi = b * _N, (b + 1) * _N
        g2 = jnp.dot(U[lo:hi, :], V[lo:hi, :].T,
                     preferred_element_type=jnp.float32)
        aa = aa_t[lo:hi, :]
        bb = jax.lax.transpose(bb_t[lo:hi, :], (1, 0))
        y = (g2 + aa) + bb
        colmin = jnp.maximum(jnp.min(y, axis=0, keepdims=True), 0.0)
        rowmin = jnp.maximum(jnp.min(y, axis=1, keepdims=True), 0.0)
        a_v = jnp.minimum(colmin, jax.lax.transpose(2.0 * d2[lo:hi, :], (1, 0)))
        b_v = jnp.minimum(rowmin, 2.0 * d1[lo:hi, :])
        out_ref[b] = (jnp.sum(a_v) + jnp.sum(b_v)) / jnp.float32(4 * _N)


def _emb_mask():
    i = jax.lax.broadcasted_iota(jnp.int32, (8, 1), 0)
    return (i >= 3).astype(jnp.float32)


def kernel(dense_t1, dense_t2, t1_mask, t2_mask, W_gat, att_src, att_dst,
           b_gat, Wd1, bd1, Wd2, bd2, Wi1, bi1, Wi2, bi2, emb_del, emb_ins):
    del t1_mask, t2_mask
    P1 = dense_t1.reshape(_B * _N, _D)
    P2 = dense_t2.reshape(_B * _N, _D)
    asrc = att_src.reshape(1, _D)
    adst = att_dst.reshape(1, _D)
    bgat = b_gat.reshape(1, _D)
    bd1r = bd1.reshape(1, _D // 2)
    bi1r = bi1.reshape(1, _D // 2)

    def fixed(shape):
        return pl.BlockSpec(shape, lambda i: (0,) * len(shape))

    out = pl.pallas_call(
        _soft_hd_kernel,
        grid=(1,),
        in_specs=[
            fixed((_B * _N, _D)),
            fixed((_B * _N, _D)),
            fixed((_D, _D)),
            fixed((1, _D)),
            fixed((1, _D)),
            fixed((1, _D)),
            fixed((_D // 2, _D)),
            fixed((1, _D // 2)),
            fixed((1, _D // 2)),
            pl.BlockSpec(memory_space=pltpu.SMEM),
            fixed((_D // 2, _D)),
            fixed((1, _D // 2)),
            fixed((1, _D // 2)),
            pl.BlockSpec(memory_space=pltpu.SMEM),
            pl.BlockSpec((8, 1), lambda i: (_N // 8 - 1, 0)),
            pl.BlockSpec((8, 1), lambda i: (_N // 8 - 1, 0)),
        ],
        out_specs=pl.BlockSpec(memory_space=pltpu.SMEM),
        out_shape=jax.ShapeDtypeStruct((_B,), jnp.float32),
    )(P1, P2, W_gat, asrc, adst, bgat,
      Wd1, bd1r, Wd2, bd2, Wi1, bi1r, Wi2, bi2, emb_del, emb_ins)
    return out


# speedup vs baseline: 26.6885x; 1.0604x over previous
import jax
import jax.numpy as jnp
from jax.experimental import pallas as pl
from jax.experimental.pallas import tpu as pltpu

_B = 2
_N = 512
_D = 128


def _emb_mask():
    i = jax.lax.broadcasted_iota(jnp.int32, (8, 1), 0)
    return (i >= 3).astype(jnp.float32)


def _soft_hd_kernel(p1_ref, p2_ref, W_ref, asrc_ref, adst_ref, bgat_ref,
                    Wd1_ref, bd1_ref, Wd2_ref, bd2_ref,
                    Wi1_ref, bi1_ref, Wi2_ref, bi2_ref,
                    edel_ref, eins_ref, out_ref):
    P1 = p1_ref[...]
    P2 = p2_ref[...]
    W = W_ref[...]
    bgat = bgat_ref[...]

    M = _B * _N
    idx = jax.lax.broadcasted_iota(jnp.int32, (M, 1), 0)
    pos = jax.lax.bitwise_and(idx, _N - 1)
    has_m = (pos >= 1).astype(jnp.float32)
    has_p = (pos <= _N - 2).astype(jnp.float32)
    neg_big = jnp.float32(-1e30)

    att = jnp.concatenate(
        [jax.lax.transpose(asrc_ref[...], (1, 0)),
         jax.lax.transpose(adst_ref[...], (1, 0))], axis=1)

    def lrelu(x):
        return jnp.where(x >= 0, x, 0.2 * x)

    def gat(p):
        h = jnp.dot(p, W.T, preferred_element_type=jnp.float32)
        e = jnp.dot(h, att, preferred_element_type=jnp.float32)
        es = e[:, 0:1]
        ed = e[:, 1:2]
        es_m = jnp.roll(es, 1, axis=0)
        es_p = jnp.roll(es, -1, axis=0)
        e_m = jnp.where(has_m > 0, lrelu(es_m + ed), neg_big)
        e_s = lrelu(es + ed)
        e_p = jnp.where(has_p > 0, lrelu(es_p + ed), neg_big)
        m = jnp.maximum(jnp.maximum(e_m, e_p), e_s)
        w_m = jnp.exp(e_m - m) * has_m
        w_s = jnp.exp(e_s - m)
        w_p = jnp.exp(e_p - m) * has_p
        inv_s = 1.0 / (w_m + w_s + w_p)
        a_m = w_m * inv_s
        a_s = w_s * inv_s
        a_p = w_p * inv_s
        h_m = jnp.roll(h, 1, axis=0)
        h_p = jnp.roll(h, -1, axis=0)
        return a_m * h_m + a_s * h + a_p * h_p + bgat

    H1 = gat(P1)
    H2 = gat(P2)

    U = jnp.concatenate([-2.0 * P1, -2.0 * H1], axis=1)
    V = jnp.concatenate([P2, H2], axis=1)
    UU = U * U
    VV = V * V
    quarter = jnp.full((2 * _D, 8), 0.25, dtype=jnp.float32)
    aa_t = jnp.dot(UU, quarter,
                   preferred_element_type=jnp.float32)[:, 0:1]
    ones_row = jnp.full((1, 2 * _D), 1.0, dtype=jnp.float32)

    def mlp_q(p, W1_ref, b1_ref):
        return jnp.maximum(
            jnp.dot(p, W1_ref[...].T, preferred_element_type=jnp.float32)
            + b1_ref[...], 0.0)

    q1 = mlp_q(P1, Wd1_ref, bd1_ref)
    q2 = mlp_q(P2, Wi1_ref, bi1_ref)
    w2d = jnp.concatenate(
        [jax.lax.transpose(Wd2_ref[...], (1, 0)),
         jnp.zeros((_D // 2, 7), jnp.float32)], axis=1)
    r1 = jnp.dot(q1, w2d, preferred_element_type=jnp.float32)[:, 0:1]
    mean_del = jnp.sum(edel_ref[...] * _emb_mask()) / 5.0
    d1 = mean_del + jnp.abs(r1 + bd2_ref[0])
    r2 = jnp.dot(Wi2_ref[...], q2.T,
                 preferred_element_type=jnp.float32)
    mean_ins = jnp.sum(eins_ref[...] * _emb_mask()) / 5.0
    d2 = mean_ins + jnp.abs(r2 + bi2_ref[0])

    for b in range(_B):
        lo, hi = b * _N, (b + 1) * _N
        g2 = jnp.dot(U[lo:hi, :], V[lo:hi, :].T,
                     preferred_element_type=jnp.float32)
        bb = jnp.dot(ones_row, VV[lo:hi, :].T,
                     preferred_element_type=jnp.float32)
        y = (g2 + aa_t[lo:hi, :]) + bb
        colmin = jnp.maximum(jnp.min(y, axis=0, keepdims=True), 0.0)
        rowmin = jnp.maximum(jnp.min(y, axis=1, keepdims=True), 0.0)
        a_v = jnp.minimum(colmin, 2.0 * d2[:, lo:hi])
        b_v = jnp.minimum(rowmin, 2.0 * d1[lo:hi, :])
        out_ref[b] = (jnp.sum(a_v) + jnp.sum(b_v)) / jnp.float32(4 * _N)


def kernel(dense_t1, dense_t2, t1_mask, t2_mask, W_gat, att_src, att_dst,
           b_gat, Wd1, bd1, Wd2, bd2, Wi1, bi1, Wi2, bi2, emb_del, emb_ins):
    del t1_mask, t2_mask
    P1 = dense_t1.reshape(_B * _N, _D)
    P2 = dense_t2.reshape(_B * _N, _D)
    asrc = att_src.reshape(1, _D)
    adst = att_dst.reshape(1, _D)
    bgat = b_gat.reshape(1, _D)
    bd1r = bd1.reshape(1, _D // 2)
    bi1r = bi1.reshape(1, _D // 2)

    def fixed(shape):
        return pl.BlockSpec(shape, lambda i: (0,) * len(shape))

    out = pl.pallas_call(
        _soft_hd_kernel,
        grid=(1,),
        in_specs=[
            fixed((_B * _N, _D)),
            fixed((_B * _N, _D)),
            fixed((_D, _D)),
            fixed((1, _D)),
            fixed((1, _D)),
            fixed((1, _D)),
            fixed((_D // 2, _D)),
            fixed((1, _D // 2)),
            fixed((1, _D // 2)),
            pl.BlockSpec(memory_space=pltpu.SMEM),
            fixed((_D // 2, _D)),
            fixed((1, _D // 2)),
            fixed((1, _D // 2)),
            pl.BlockSpec(memory_space=pltpu.SMEM),
            pl.BlockSpec((8, 1), lambda i: (_N // 8 - 1, 0)),
            pl.BlockSpec((8, 1), lambda i: (_N // 8 - 1, 0)),
        ],
        out_specs=pl.BlockSpec(memory_space=pltpu.SMEM),
        out_shape=jax.ShapeDtypeStruct((_B,), jnp.float32),
    )(P1, P2, W_gat, asrc, adst, bgat,
      Wd1, bd1r, Wd2, bd2, Wi1, bi1r, Wi2, bi2, emb_del, emb_ins)
    return out


# speedup vs baseline: 30.9059x; 1.1580x over previous
import jax
import jax.numpy as jnp
from jax.experimental import pallas as pl
from jax.experimental.pallas import tpu as pltpu

_B = 2
_N = 512
_D = 128


def _soft_hd_kernel(p1_ref, p2_ref, wp_ref, s_ref, out_ref):
    P1 = p1_ref[...]
    P2 = p2_ref[...]
    W = wp_ref[0:_D, :]
    Wd1 = wp_ref[_D:_D + 64, :]
    Wi1 = wp_ref[_D + 64:_D + 128, :]
    asrc = wp_ref[256:257, :]
    adst = wp_ref[264:265, :]
    bgat = wp_ref[272:273, :]
    bd1 = wp_ref[280:281, 0:64]
    bi1 = wp_ref[288:289, 0:64]
    w2d_row = wp_ref[296:297, 0:64]
    w2i_row = wp_ref[297:298, 0:64]
    bd2 = s_ref[0]
    bi2 = s_ref[1]
    mean_del = (s_ref[2] + s_ref[3] + s_ref[4] + s_ref[5] + s_ref[6]) / 5.0
    mean_ins = (s_ref[7] + s_ref[8] + s_ref[9] + s_ref[10] + s_ref[11]) / 5.0

    M = _B * _N
    idx = jax.lax.broadcasted_iota(jnp.int32, (M, 1), 0)
    pos = jax.lax.bitwise_and(idx, _N - 1)
    has_m = (pos >= 1).astype(jnp.float32)
    has_p = (pos <= _N - 2).astype(jnp.float32)
    neg_big = jnp.float32(-1e30)

    att = jnp.concatenate(
        [jax.lax.transpose(asrc, (1, 0)),
         jax.lax.transpose(adst, (1, 0))], axis=1)

    def lrelu(x):
        return jnp.where(x >= 0, x, 0.2 * x)

    def gat(p):
        h = jnp.dot(p, W.T, preferred_element_type=jnp.float32)
        e = jnp.dot(h, att, preferred_element_type=jnp.float32)
        es = e[:, 0:1]
        ed = e[:, 1:2]
        es_m = jnp.roll(es, 1, axis=0)
        es_p = jnp.roll(es, -1, axis=0)
        e_m = jnp.where(has_m > 0, lrelu(es_m + ed), neg_big)
        e_s = lrelu(es + ed)
        e_p = jnp.where(has_p > 0, lrelu(es_p + ed), neg_big)
        m = jnp.maximum(jnp.maximum(e_m, e_p), e_s)
        w_m = jnp.exp(e_m - m) * has_m
        w_s = jnp.exp(e_s - m)
        w_p = jnp.exp(e_p - m) * has_p
        inv_s = 1.0 / (w_m + w_s + w_p)
        a_m = w_m * inv_s
        a_s = w_s * inv_s
        a_p = w_p * inv_s
        h_m = jnp.roll(h, 1, axis=0)
        h_p = jnp.roll(h, -1, axis=0)
        return a_m * h_m + a_s * h + a_p * h_p + bgat

    H1 = gat(P1)
    H2 = gat(P2)

    U = jnp.concatenate([-2.0 * P1, -2.0 * H1], axis=1)
    V = jnp.concatenate([P2, H2], axis=1)
    UU = U * U
    VV = V * V
    quarter = jnp.full((2 * _D, 8), 0.25, dtype=jnp.float32)
    aa_t = jnp.dot(UU, quarter,
                   preferred_element_type=jnp.float32)[:, 0:1]
    ones_row = jnp.full((1, 2 * _D), 1.0, dtype=jnp.float32)

    def mlp_q(p, W1, b1):
        return jnp.maximum(
            jnp.dot(p, W1.T, preferred_element_type=jnp.float32) + b1,
            0.0)

    q1 = mlp_q(P1, Wd1, bd1)
    q2 = mlp_q(P2, Wi1, bi1)
    w2d = jnp.concatenate(
        [jax.lax.transpose(w2d_row, (1, 0)),
         jnp.zeros((_D // 2, 7), jnp.float32)], axis=1)
    r1 = jnp.dot(q1, w2d, preferred_element_type=jnp.float32)[:, 0:1]
    d1 = mean_del + jnp.abs(r1 + bd2)
    r2 = jnp.dot(w2i_row, q2.T,
                 preferred_element_type=jnp.float32)
    d2 = mean_ins + jnp.abs(r2 + bi2)

    for b in range(_B):
        lo, hi = b * _N, (b + 1) * _N
        g2 = jnp.dot(U[lo:hi, :], V[lo:hi, :].T,
                     preferred_element_type=jnp.float32)
        bb = jnp.dot(ones_row, VV[lo:hi, :].T,
                     preferred_element_type=jnp.float32)
        y = (g2 + aa_t[lo:hi, :]) + bb
        colmin = jnp.maximum(jnp.min(y, axis=0, keepdims=True), 0.0)
        rowmin = jnp.maximum(jnp.min(y, axis=1, keepdims=True), 0.0)
        a_v = jnp.minimum(colmin, 2.0 * d2[:, lo:hi])
        b_v = jnp.minimum(rowmin, 2.0 * d1[lo:hi, :])
        out_ref[b] = (jnp.sum(a_v) + jnp.sum(b_v)) / jnp.float32(4 * _N)


def kernel(dense_t1, dense_t2, t1_mask, t2_mask, W_gat, att_src, att_dst,
           b_gat, Wd1, bd1, Wd2, bd2, Wi1, bi1, Wi2, bi2, emb_del, emb_ins):
    del t1_mask, t2_mask
    P1 = dense_t1.reshape(_B * _N, _D)
    P2 = dense_t2.reshape(_B * _N, _D)
    z7 = jnp.zeros((7 * _D,), jnp.float32)
    z64 = jnp.zeros((_D // 2,), jnp.float32)
    z6 = jnp.zeros((6 * _D,), jnp.float32)
    packed = jnp.concatenate([
        W_gat.ravel(), Wd1.ravel(), Wi1.ravel(),
        att_src, z7, att_dst, z7, b_gat, z7,
        bd1, z64, z7, bi1, z64, z7,
        Wd2.ravel(), z64, Wi2.ravel(), z64, z6,
    ]).reshape(304, _D)
    scal = jnp.concatenate(
        [bd2, bi2, emb_del[_N - 5:_N, 0], emb_ins[_N - 5:_N, 0]])

    def fixed(shape):
        return pl.BlockSpec(shape, lambda i: (0,) * len(shape))

    out = pl.pallas_call(
        _soft_hd_kernel,
        grid=(1,),
        in_specs=[fixed((_B * _N, _D)), fixed((_B * _N, _D)),
                  fixed((304, _D)), pl.BlockSpec(memory_space=pltpu.SMEM)],
        out_specs=pl.BlockSpec(memory_space=pltpu.SMEM),
        out_shape=jax.ShapeDtypeStruct((_B,), jnp.float32),
    )(P1, P2, packed, scal)
    return out
